# Initial kernel scaffold; baseline (speedup 1.0000x reference)
import jax
import jax.numpy as jnp
from jax.experimental import pallas as pl


def kernel(x, edge_index_ct, edge_feat_ct, edge_index_cb, edge_feat_cb, W0, b0, W1, b1):
    raise NotImplementedError("write your pallas kernel here")


# speedup vs baseline: 2.5483x; 2.5483x over previous
import functools

import jax
import jax.numpy as jnp
from jax import lax
from jax.experimental import pallas as pl
from jax.experimental.pallas import tpu as pltpu
from jax.experimental.pallas import tpu_sc as plsc

N = 10000
E = 320000
D = 128
NS = 16
LANES = 16
B = 80
E_PER_TILE = E // NS
N_CHUNKS = E_PER_TILE // B
N_PAD = 10240
ROWS_PER_TILE = N_PAD // NS


def _sc_body(h, src_ct, dst_ct, feat_ct, src_cb, dst_cb, feat_cb, zeros,
             out_ct, out_cb, agg, src_v, dst_v, gath_v, feat_v, sem):
    cid = lax.axis_index("c")
    sid = lax.axis_index("s")

    def run(src_h, dst_h, feat_h, out_h):
        row0 = sid * ROWS_PER_TILE
        pltpu.sync_copy(zeros.at[pl.ds(row0, ROWS_PER_TILE)],
                        agg.at[pl.ds(row0, ROWS_PER_TILE)])
        plsc.subcore_barrier()

        base0 = sid * E_PER_TILE

        def chunk(c, carry):
            base = base0 + c * B
            pltpu.sync_copy(src_h.at[pl.ds(base, B)], src_v)
            pltpu.sync_copy(dst_h.at[pl.ds(base, B)], dst_v)
            pltpu.async_copy(h.at[src_v], gath_v, sem).wait()
            pltpu.sync_copy(feat_h.at[pl.ds(base, B)], feat_v)

            def row(i, c2):
                for j in range(D // LANES):
                    s = pl.ds(j * LANES, LANES)
                    feat_v[i, s] = feat_v[i, s] * gath_v[i, s]
                return c2

            lax.fori_loop(0, B, row, 0)
            pltpu.sync_copy(feat_v, agg.at[dst_v], add=True)
            return carry

        lax.fori_loop(0, N_CHUNKS, chunk, 0)
        plsc.subcore_barrier()
        pltpu.sync_copy(agg.at[pl.ds(row0, ROWS_PER_TILE)],
                        out_h.at[pl.ds(row0, ROWS_PER_TILE)])

    @pl.when(cid == 0)
    def _():
        run(src_ct, dst_ct, feat_ct, out_ct)

    @pl.when(cid == 1)
    def _():
        run(src_cb, dst_cb, feat_cb, out_cb)


_sc_call = pl.kernel(
    _sc_body,
    out_type=[jax.ShapeDtypeStruct((N_PAD, D), jnp.float32)] * 2,
    mesh=plsc.VectorSubcoreMesh(core_axis_name="c", subcore_axis_name="s"),
    scratch_types=[
        pltpu.VMEM_SHARED((N_PAD, D), jnp.float32),
        pltpu.VMEM((B,), jnp.int32),
        pltpu.VMEM((B,), jnp.int32),
        pltpu.VMEM((B, D), jnp.float32),
        pltpu.VMEM((B, D), jnp.float32),
        pltpu.SemaphoreType.DMA,
    ],
)


def _tc_body(h_ref, act_ref, acb_ref, wa_ref, wb_ref, bias_ref, out_ref):
    acc = jnp.dot(act_ref[...], wa_ref[...], preferred_element_type=jnp.float32)
    acc = acc + jnp.dot(acb_ref[...], wb_ref[...], preferred_element_type=jnp.float32)
    acc = acc + bias_ref[...]
    g = 0.5 * acc * (1.0 + lax.erf(acc * (2.0 ** -0.5)))
    out_ref[...] = h_ref[...] + g


_TC_R = 2000
_tc_call = pl.pallas_call(
    _tc_body,
    grid=(N // _TC_R,),
    in_specs=[
        pl.BlockSpec((_TC_R, D), lambda i: (i, 0)),
        pl.BlockSpec((_TC_R, D), lambda i: (i, 0)),
        pl.BlockSpec((_TC_R, D), lambda i: (i, 0)),
        pl.BlockSpec((D, D), lambda i: (0, 0)),
        pl.BlockSpec((D, D), lambda i: (0, 0)),
        pl.BlockSpec((1, D), lambda i: (0, 0)),
    ],
    out_specs=pl.BlockSpec((_TC_R, D), lambda i: (i, 0)),
    out_shape=jax.ShapeDtypeStruct((N, D), jnp.float32),
)


def kernel(x, edge_index_ct, edge_feat_ct, edge_index_cb, edge_feat_cb,
           W0, b0, W1, b1):
    src_ct, dst_ct = edge_index_ct[0], edge_index_ct[1]
    src_cb, dst_cb = edge_index_cb[0], edge_index_cb[1]
    zeros = jnp.zeros((N_PAD, D), jnp.float32)
    h = x
    for (W, b) in ((W0, b0), (W1, b1)):
        wt = jnp.transpose(W)
        wa, wb = wt[:D], wt[D:]
        agg_ct, agg_cb = _sc_call(h, src_ct, dst_ct, edge_feat_ct,
                                  src_cb, dst_cb, edge_feat_cb, zeros)
        h = _tc_call(h, agg_ct, agg_cb, wa, wb, b.reshape(1, D))
    return h


# speedup vs baseline: 2.6820x; 1.0525x over previous
import jax
import jax.numpy as jnp
from jax import lax
from jax.experimental import pallas as pl
from jax.experimental.pallas import tpu as pltpu
from jax.experimental.pallas import tpu_sc as plsc

N = 10000
E = 320000
D = 128
NS = 16
LANES = 16
B = 80
E_PER_TILE = E // NS
NCH_REAL = E_PER_TILE // B
NCH = 256
G = 16
NG = NCH // G
N_PAD = 10112
ROWS_PER_TILE = N_PAD // NS


def _sc_body(h, src_ct, dst_ct, feat_ct, src_cb, dst_cb, feat_cb, zeros,
             out_ct, out_cb,
             agg, isrc_a, idst_a, isrc_b, idst_b,
             gath0, gath1, feat0, feat1,
             isem_a, isem_b, sem0, sem1):
    cid = lax.axis_index("c")
    sid = lax.axis_index("s")

    def run(src2, dst2, feat_h, out_h):
        row0 = sid * ROWS_PER_TILE
        pltpu.sync_copy(zeros.at[pl.ds(row0, ROWS_PER_TILE)],
                        agg.at[pl.ds(row0, ROWS_PER_TILE)])

        grp0 = sid * NCH

        def idx_slices(g):
            r = pl.ds(grp0 + g * G, G)
            return src2.at[r], dst2.at[r]

        def issue_idx(g, isrc, idst, isem):
            s, d2 = idx_slices(g)
            pltpu.async_copy(s, isrc, isem)
            pltpu.async_copy(d2, idst, isem)

        def wait_idx(g, isrc, idst, isem):
            s, d2 = idx_slices(g)
            pltpu.make_async_copy(s, isrc, isem).wait()
            pltpu.make_async_copy(d2, idst, isem).wait()

        base0 = sid * E_PER_TILE

        def feat_slice(c):
            off = jnp.where(c < NCH_REAL, c * B, 0)
            return feat_h.at[pl.ds(base0 + off, B)]

        def issue(c, k, isrc, slot_g, slot_f, sem):
            pltpu.async_copy(h.at[isrc.at[k]], slot_g, sem)
            pltpu.async_copy(feat_slice(c), slot_f, sem)

        def process(c, k, isrc, idst, slot_g, slot_f, sem):
            pltpu.make_async_copy(h.at[isrc.at[k]], slot_g, sem).wait()
            pltpu.make_async_copy(feat_slice(c), slot_f, sem).wait()

            def row(i, c2):
                for j in range(D // LANES):
                    s = pl.ds(j * LANES, LANES)
                    slot_f[i, s] = slot_f[i, s] * slot_g[i, s]
                return c2

            lax.fori_loop(0, B, row, 0)
            pltpu.sync_copy(slot_f, agg.at[idst.at[k]], add=True)

        issue_idx(0, isrc_a, idst_a, isem_a)
        plsc.subcore_barrier()

        def group_body(g, isrc, idst, isem):
            c0 = g * G
            issue(c0, 0, isrc, gath0, feat0, sem0)

            def it(i, carry):
                k = 2 * i
                issue(c0 + k + 1, k + 1, isrc, gath1, feat1, sem1)
                process(c0 + k, k, isrc, idst, gath0, feat0, sem0)

                @pl.when(k + 2 < G)
                def _():
                    issue(c0 + k + 2, k + 2, isrc, gath0, feat0, sem0)

                process(c0 + k + 1, k + 1, isrc, idst, gath1, feat1, sem1)
                return carry

            lax.fori_loop(0, G // 2, it, 0)

        def pair(p, carry):
            g = 2 * p
            wait_idx(g, isrc_a, idst_a, isem_a)
            issue_idx(g + 1, isrc_b, idst_b, isem_b)
            group_body(g, isrc_a, idst_a, isem_a)
            wait_idx(g + 1, isrc_b, idst_b, isem_b)

            @pl.when(g + 2 < NG)
            def _():
                issue_idx(g + 2, isrc_a, idst_a, isem_a)

            group_body(g + 1, isrc_b, idst_b, isem_b)
            return carry

        lax.fori_loop(0, NG // 2, pair, 0)

        plsc.subcore_barrier()
        pltpu.sync_copy(agg.at[pl.ds(row0, ROWS_PER_TILE)],
                        out_h.at[pl.ds(row0, ROWS_PER_TILE)])

    @pl.when(cid == 0)
    def _():
        run(src_ct, dst_ct, feat_ct, out_ct)

    @pl.when(cid == 1)
    def _():
        run(src_cb, dst_cb, feat_cb, out_cb)


_sc_call = pl.kernel(
    _sc_body,
    out_type=[jax.ShapeDtypeStruct((N_PAD, D), jnp.float32)] * 2,
    mesh=plsc.VectorSubcoreMesh(core_axis_name="c", subcore_axis_name="s"),
    scratch_types=[
        pltpu.VMEM_SHARED((N_PAD, D), jnp.float32),
        pltpu.VMEM((G, B), jnp.int32),
        pltpu.VMEM((G, B), jnp.int32),
        pltpu.VMEM((G, B), jnp.int32),
        pltpu.VMEM((G, B), jnp.int32),
        pltpu.VMEM((B, D), jnp.float32),
        pltpu.VMEM((B, D), jnp.float32),
        pltpu.VMEM((B, D), jnp.float32),
        pltpu.VMEM((B, D), jnp.float32),
        pltpu.SemaphoreType.DMA,
        pltpu.SemaphoreType.DMA,
        pltpu.SemaphoreType.DMA,
        pltpu.SemaphoreType.DMA,
    ],
)


def _pack_idx(idx, synth):
    a = idx.reshape(NS, NCH_REAL, B)
    pad = jnp.broadcast_to(synth.reshape(1, NCH - NCH_REAL, B),
                           (NS, NCH - NCH_REAL, B))
    return jnp.concatenate([a, pad], axis=1).reshape(NS * NCH, B)


def _tc_body(h_ref, act_ref, acb_ref, wa_ref, wb_ref, bias_ref, out_ref):
    acc = jnp.dot(act_ref[...], wa_ref[...], preferred_element_type=jnp.float32)
    acc = acc + jnp.dot(acb_ref[...], wb_ref[...], preferred_element_type=jnp.float32)
    acc = acc + bias_ref[...]
    g = 0.5 * acc * (1.0 + lax.erf(acc * (2.0 ** -0.5)))
    out_ref[...] = h_ref[...] + g


_TC_R = 2000
_tc_call = pl.pallas_call(
    _tc_body,
    grid=(N // _TC_R,),
    in_specs=[
        pl.BlockSpec((_TC_R, D), lambda i: (i, 0)),
        pl.BlockSpec((_TC_R, D), lambda i: (i, 0)),
        pl.BlockSpec((_TC_R, D), lambda i: (i, 0)),
        pl.BlockSpec((D, D), lambda i: (0, 0)),
        pl.BlockSpec((D, D), lambda i: (0, 0)),
        pl.BlockSpec((1, D), lambda i: (0, 0)),
    ],
    out_specs=pl.BlockSpec((_TC_R, D), lambda i: (i, 0)),
    out_shape=jax.ShapeDtypeStruct((N, D), jnp.float32),
)


def kernel(x, edge_index_ct, edge_feat_ct, edge_index_cb, edge_feat_cb,
           W0, b0, W1, b1):
    n_syn = (NCH - NCH_REAL) * B
    synth_src = jnp.zeros((n_syn,), jnp.int32)
    synth_dst = N + (jnp.arange(n_syn, dtype=jnp.int32) % (N_PAD - N))
    src_ct = _pack_idx(edge_index_ct[0], synth_src)
    dst_ct = _pack_idx(edge_index_ct[1], synth_dst)
    src_cb = _pack_idx(edge_index_cb[0], synth_src)
    dst_cb = _pack_idx(edge_index_cb[1], synth_dst)
    zeros = jnp.zeros((N_PAD, D), jnp.float32)
    h = x
    for (W, b) in ((W0, b0), (W1, b1)):
        wt = jnp.transpose(W)
        wa, wb = wt[:D], wt[D:]
        agg_ct, agg_cb = _sc_call(h, src_ct, dst_ct, edge_feat_ct,
                                  src_cb, dst_cb, edge_feat_cb, zeros)
        h = _tc_call(h, agg_ct, agg_cb, wa, wb, b.reshape(1, D))
    return h


# speedup vs baseline: 2.6855x; 1.0013x over previous
import jax
import jax.numpy as jnp
from jax import lax
from jax.experimental import pallas as pl
from jax.experimental.pallas import tpu as pltpu
from jax.experimental.pallas import tpu_sc as plsc

N = 10000
E = 320000
D = 128
NS = 16
LANES = 16
B = 80
E_PER_TILE = E // NS
NCH_REAL = E_PER_TILE // B
NCH = 256
G = 16
NG = NCH // G
N_PAD = 10112
ROWS_PER_TILE = N_PAD // NS


def _sc_body(h, src_ct, dst_ct, feat_ct, src_cb, dst_cb, feat_cb, zeros,
             out_ct, out_cb,
             agg, isrc_a, idst_a, isrc_b, idst_b,
             gath0, gath1, feat0, feat1,
             isem_a, isem_b, sem0, sem1):
    cid = lax.axis_index("c")
    sid = lax.axis_index("s")

    def run(src2, dst2, feat_h, out_h):
        row0 = sid * ROWS_PER_TILE
        pltpu.sync_copy(zeros.at[pl.ds(row0, ROWS_PER_TILE)],
                        agg.at[pl.ds(row0, ROWS_PER_TILE)])

        grp0 = sid * NCH

        def idx_slices(g):
            r = pl.ds(grp0 + g * G, G)
            return src2.at[r], dst2.at[r]

        def issue_idx(g, isrc, idst, isem):
            s, d2 = idx_slices(g)
            pltpu.async_copy(s, isrc, isem)
            pltpu.async_copy(d2, idst, isem)

        def wait_idx(g, isrc, idst, isem):
            s, d2 = idx_slices(g)
            pltpu.make_async_copy(s, isrc, isem).wait()
            pltpu.make_async_copy(d2, idst, isem).wait()

        base0 = sid * E_PER_TILE

        def feat_slice(c):
            off = jnp.where(c < NCH_REAL, c * B, 0)
            return feat_h.at[pl.ds(base0 + off, B)]

        H2 = B // 2

        def issue(c, k, isrc, slot_g, slot_f, sem):
            pltpu.async_copy(h.at[isrc.at[k, pl.ds(0, H2)]], slot_g.at[pl.ds(0, H2)], sem)
            pltpu.async_copy(h.at[isrc.at[k, pl.ds(H2, H2)]], slot_g.at[pl.ds(H2, H2)], sem)
            pltpu.async_copy(feat_slice(c), slot_f, sem)

        def process(c, k, isrc, idst, slot_g, slot_f, sem):
            pltpu.make_async_copy(h.at[isrc.at[k, pl.ds(0, H2)]], slot_g.at[pl.ds(0, H2)], sem).wait()
            pltpu.make_async_copy(h.at[isrc.at[k, pl.ds(H2, H2)]], slot_g.at[pl.ds(H2, H2)], sem).wait()
            pltpu.make_async_copy(feat_slice(c), slot_f, sem).wait()

            def row(i, c2):
                for j in range(D // LANES):
                    s = pl.ds(j * LANES, LANES)
                    slot_f[i, s] = slot_f[i, s] * slot_g[i, s]
                return c2

            lax.fori_loop(0, B, row, 0)
            pltpu.sync_copy(slot_f, agg.at[idst.at[k]], add=True)

        issue_idx(0, isrc_a, idst_a, isem_a)
        plsc.subcore_barrier()

        def group_body(g, isrc, idst, isem):
            c0 = g * G
            issue(c0, 0, isrc, gath0, feat0, sem0)

            def it(i, carry):
                k = 2 * i
                issue(c0 + k + 1, k + 1, isrc, gath1, feat1, sem1)
                process(c0 + k, k, isrc, idst, gath0, feat0, sem0)

                @pl.when(k + 2 < G)
                def _():
                    issue(c0 + k + 2, k + 2, isrc, gath0, feat0, sem0)

                process(c0 + k + 1, k + 1, isrc, idst, gath1, feat1, sem1)
                return carry

            lax.fori_loop(0, G // 2, it, 0)

        def pair(p, carry):
            g = 2 * p
            wait_idx(g, isrc_a, idst_a, isem_a)
            issue_idx(g + 1, isrc_b, idst_b, isem_b)
            group_body(g, isrc_a, idst_a, isem_a)
            wait_idx(g + 1, isrc_b, idst_b, isem_b)

            @pl.when(g + 2 < NG)
            def _():
                issue_idx(g + 2, isrc_a, idst_a, isem_a)

            group_body(g + 1, isrc_b, idst_b, isem_b)
            return carry

        lax.fori_loop(0, NG // 2, pair, 0)

        plsc.subcore_barrier()
        pltpu.sync_copy(agg.at[pl.ds(row0, ROWS_PER_TILE)],
                        out_h.at[pl.ds(row0, ROWS_PER_TILE)])

    @pl.when(cid == 0)
    def _():
        run(src_ct, dst_ct, feat_ct, out_ct)

    @pl.when(cid == 1)
    def _():
        run(src_cb, dst_cb, feat_cb, out_cb)


_sc_call = pl.kernel(
    _sc_body,
    out_type=[jax.ShapeDtypeStruct((N_PAD, D), jnp.float32)] * 2,
    mesh=plsc.VectorSubcoreMesh(core_axis_name="c", subcore_axis_name="s"),
    scratch_types=[
        pltpu.VMEM_SHARED((N_PAD, D), jnp.float32),
        pltpu.VMEM((G, B), jnp.int32),
        pltpu.VMEM((G, B), jnp.int32),
        pltpu.VMEM((G, B), jnp.int32),
        pltpu.VMEM((G, B), jnp.int32),
        pltpu.VMEM((B, D), jnp.float32),
        pltpu.VMEM((B, D), jnp.float32),
        pltpu.VMEM((B, D), jnp.float32),
        pltpu.VMEM((B, D), jnp.float32),
        pltpu.SemaphoreType.DMA,
        pltpu.SemaphoreType.DMA,
        pltpu.SemaphoreType.DMA,
        pltpu.SemaphoreType.DMA,
    ],
)


def _pack_idx(idx, synth):
    a = idx.reshape(NS, NCH_REAL, B)
    pad = jnp.broadcast_to(synth.reshape(1, NCH - NCH_REAL, B),
                           (NS, NCH - NCH_REAL, B))
    return jnp.concatenate([a, pad], axis=1).reshape(NS * NCH, B)


def _tc_body(h_ref, act_ref, acb_ref, wa_ref, wb_ref, bias_ref, out_ref):
    acc = jnp.dot(act_ref[...], wa_ref[...], preferred_element_type=jnp.float32)
    acc = acc + jnp.dot(acb_ref[...], wb_ref[...], preferred_element_type=jnp.float32)
    acc = acc + bias_ref[...]
    g = 0.5 * acc * (1.0 + lax.erf(acc * (2.0 ** -0.5)))
    out_ref[...] = h_ref[...] + g


_TC_R = 2000
_tc_call = pl.pallas_call(
    _tc_body,
    grid=(N // _TC_R,),
    in_specs=[
        pl.BlockSpec((_TC_R, D), lambda i: (i, 0)),
        pl.BlockSpec((_TC_R, D), lambda i: (i, 0)),
        pl.BlockSpec((_TC_R, D), lambda i: (i, 0)),
        pl.BlockSpec((D, D), lambda i: (0, 0)),
        pl.BlockSpec((D, D), lambda i: (0, 0)),
        pl.BlockSpec((1, D), lambda i: (0, 0)),
    ],
    out_specs=pl.BlockSpec((_TC_R, D), lambda i: (i, 0)),
    out_shape=jax.ShapeDtypeStruct((N, D), jnp.float32),
)


def kernel(x, edge_index_ct, edge_feat_ct, edge_index_cb, edge_feat_cb,
           W0, b0, W1, b1):
    n_syn = (NCH - NCH_REAL) * B
    synth_src = jnp.zeros((n_syn,), jnp.int32)
    synth_dst = N + (jnp.arange(n_syn, dtype=jnp.int32) % (N_PAD - N))
    src_ct = _pack_idx(edge_index_ct[0], synth_src)
    dst_ct = _pack_idx(edge_index_ct[1], synth_dst)
    src_cb = _pack_idx(edge_index_cb[0], synth_src)
    dst_cb = _pack_idx(edge_index_cb[1], synth_dst)
    zeros = jnp.zeros((N_PAD, D), jnp.float32)
    h = x
    for (W, b) in ((W0, b0), (W1, b1)):
        wt = jnp.transpose(W)
        wa, wb = wt[:D], wt[D:]
        agg_ct, agg_cb = _sc_call(h, src_ct, dst_ct, edge_feat_ct,
                                  src_cb, dst_cb, edge_feat_cb, zeros)
        h = _tc_call(h, agg_ct, agg_cb, wa, wb, b.reshape(1, D))
    return h
